# baseline (speedup 1.0000x reference)
import functools

import jax
import jax.numpy as jnp
from jax import lax
from jax.experimental import pallas as pl
from jax.experimental.pallas import tpu as pltpu
from jax.experimental.pallas import tpu_sc as plsc


def _sc_gather_rows(ids, table):
    (B,) = ids.shape
    _, H = table.shape
    mesh = plsc.VectorSubcoreMesh(core_axis_name="c", subcore_axis_name="s")

    @functools.partial(
        pl.kernel,
        mesh=mesh,
        out_type=jax.ShapeDtypeStruct((B, H), jnp.float32),
        scratch_types=[
            pltpu.VMEM((B,), jnp.int32),
            pltpu.VMEM((B, H), jnp.float32),
            pltpu.SemaphoreType.DMA,
        ],
    )
    def k(ids_hbm, table_hbm, out_hbm, idx_v, rows_v, sem):
        wid = lax.axis_index("s") * 2 + lax.axis_index("c")

        @pl.when(wid == 0)
        def _():
            pltpu.sync_copy(ids_hbm, idx_v)
            pltpu.async_copy(table_hbm.at[idx_v], rows_v, sem).wait()
            pltpu.sync_copy(rows_v, out_hbm)

    return k(ids, table)


def _tc_gated_add(x2d, embedding, gate2d, rows, B, P, H):
    def body(x_ref, emb_ref, gate_ref, row_ref, o_ref):
        g = jnp.tanh(gate_ref[...])
        o_ref[...] = x_ref[...] + g * (emb_ref[...] + row_ref[0])

    return pl.pallas_call(
        body,
        grid=(B,),
        in_specs=[
            pl.BlockSpec((P, H), lambda b: (b, 0)),
            pl.BlockSpec((P, H), lambda b: (0, 0)),
            pl.BlockSpec((1, 1), lambda b: (0, 0)),
            pl.BlockSpec((1, 1, H), lambda b: (b, 0, 0)),
        ],
        out_specs=pl.BlockSpec((P, H), lambda b: (b, 0)),
        out_shape=jax.ShapeDtypeStruct((B * P, H), jnp.float32),
    )(x2d, embedding, gate2d, rows.reshape(B, 1, H))


def kernel(x, tile_ids, embedding, gate, tile_embedding_table):
    B, P, H = x.shape
    ids = tile_ids.reshape(B).astype(jnp.int32)
    rows = _sc_gather_rows(ids, tile_embedding_table)
    out2d = _tc_gated_add(
        x.reshape(B * P, H), embedding, gate.reshape(1, 1), rows, B, P, H
    )
    return out2d.reshape(B, P, H)
